# baseline (speedup 1.0000x reference)
import functools

import jax
import jax.numpy as jnp
from jax import lax
from jax.experimental import pallas as pl
from jax.experimental.pallas import tpu as pltpu
from jax.experimental.pallas import tpu_sc as plsc

_NC = 2
_NS = 16
_CHUNK = 128


def _round_up(v, m):
    return (v + m - 1) // m * m


def _make_sc_segment_sum(n_pad, d_tot, chunks_per_worker):
    nw = _NC * _NS
    rows_per_tile = n_pad // _NS
    epw = chunks_per_worker * _CHUNK
    mesh = plsc.VectorSubcoreMesh(core_axis_name="c", subcore_axis_name="s")

    @functools.partial(
        pl.kernel,
        out_type=jax.ShapeDtypeStruct((_NC * n_pad, d_tot), jnp.float32),
        mesh=mesh,
        compiler_params=pltpu.CompilerParams(use_tc_tiling_on_sc=False),
        scratch_types=[
            pltpu.VMEM((2, _CHUNK), jnp.int32),
            pltpu.VMEM((_CHUNK, d_tot), jnp.float32),
            pltpu.VMEM_SHARED((n_pad, d_tot), jnp.float32),
            pltpu.SemaphoreType.DMA,
        ],
    )
    def sc_kernel(rows_hbm, edges_hbm, zeros_hbm, out_hbm,
                  idx_a, gat_v, acc_sh, sem_g):
        cid = lax.axis_index("c")
        sid = lax.axis_index("s")
        wid = sid * _NC + cid
        r0 = sid * rows_per_tile
        n_sub = rows_per_tile // _CHUNK
        pltpu.sync_copy(zeros_hbm, gat_v)
        for j in range(n_sub):
            pltpu.sync_copy(
                gat_v, acc_sh.at[pl.ds(r0 + j * _CHUNK, _CHUNK)])
        plsc.subcore_barrier()

        def body(i, carry):
            off = wid * epw + i * _CHUNK
            pltpu.sync_copy(edges_hbm.at[:, pl.ds(off, _CHUNK)], idx_a)
            pltpu.async_copy(rows_hbm.at[idx_a.at[0]], gat_v, sem_g).wait()
            pltpu.sync_copy(gat_v, acc_sh.at[idx_a.at[1]], add=True)
            return carry

        lax.fori_loop(0, chunks_per_worker, body, 0)
        plsc.subcore_barrier()
        for j in range(n_sub):
            pltpu.sync_copy(
                acc_sh.at[pl.ds(r0 + j * _CHUNK, _CHUNK)], gat_v)
            pltpu.sync_copy(
                gat_v,
                out_hbm.at[pl.ds(cid * n_pad + r0 + j * _CHUNK, _CHUNK)])

    return sc_kernel


def _tca_body(res1_ref, x_ref, s_ref, w1r_ref, w1n_ref, b1_ref, w2n_ref,
              h_ref, p_ref, *, blk, d):
    i = pl.program_id(0)
    s = s_ref[0] + s_ref[1]
    deg = jnp.maximum(s[:, d:d + 1], 1.0)
    agg = s[:, :d] / deg
    z = (jnp.dot(x_ref[:, :d], w1r_ref[...], preferred_element_type=jnp.float32)
         + jnp.dot(agg, w1n_ref[...], preferred_element_type=jnp.float32)
         + b1_ref[...])
    rows = i * blk + lax.broadcasted_iota(jnp.int32, (blk, 1), 0)
    h = jnp.where(rows < res1_ref[0], jnp.maximum(z, 0.0), 0.0)
    h_ref[...] = h
    p = jnp.dot(h, w2n_ref[...], preferred_element_type=jnp.float32)
    p_ref[...] = jnp.concatenate(
        [p, jnp.ones((blk, 16), jnp.float32)], axis=1)


def _tcb_body(res2_ref, h_ref, s_ref, w2r_ref, b2_ref, o_ref, *, blk, c):
    i = pl.program_id(0)
    s = s_ref[0] + s_ref[1]
    agg = s[:, :c] / jnp.maximum(s[:, c:c + 1], 1.0)
    z = (jnp.dot(h_ref[...], w2r_ref[...], preferred_element_type=jnp.float32)
         + agg + b2_ref[...])
    rows = i * blk + lax.broadcasted_iota(jnp.int32, (blk, 1), 0)
    z = jnp.where(rows < res2_ref[0], z, 0.0)
    m = jnp.max(z, axis=1, keepdims=True)
    e = jnp.exp(z - m)
    o_ref[...] = z - m - jnp.log(jnp.sum(e, axis=1, keepdims=True))


def kernel(x, edge_index1, res_size1, edge_index2, res_size2,
           W1_root, W1_neigh, b1, W2_root, W2_neigh, b2):
    n, d = x.shape
    hdim = W1_root.shape[1]
    cdim = W2_root.shape[1]
    e = edge_index1.shape[1]
    blk = 512
    n_pad = _round_up(n + 1, blk)
    nw = _NC * _NS
    cpw = _round_up(-(-e // (nw * _CHUNK)), 2)
    e_pad = cpw * nw * _CHUNK
    e_alloc = e_pad + _CHUNK
    d1 = d + 16
    d2 = cdim + 16

    xpad = jnp.pad(
        jnp.concatenate([x, jnp.ones((n, 16), jnp.float32)], axis=1),
        ((0, n_pad - n), (0, 0)))
    ei1 = edge_index1.astype(jnp.int32)
    ei2 = edge_index2.astype(jnp.int32)
    edges1 = jnp.stack([jnp.pad(ei1[0], (0, e_alloc - e)),
                        jnp.pad(ei1[1], (0, e_alloc - e), constant_values=n)])
    edges2 = jnp.stack([jnp.pad(ei2[0], (0, e_alloc - e)),
                        jnp.pad(ei2[1], (0, e_alloc - e), constant_values=n)])
    z1 = jnp.zeros((_CHUNK, d1), jnp.float32)
    z2 = jnp.zeros((_CHUNK, d2), jnp.float32)
    res1 = jnp.asarray(res_size1, jnp.int32).reshape(1)
    res2 = jnp.asarray(res_size2, jnp.int32).reshape(1)

    s1 = _make_sc_segment_sum(n_pad, d1, cpw)(xpad, edges1, z1)
    s1 = s1.reshape(_NC, n_pad, d1)

    grid = (n_pad // blk,)
    h, p = pl.pallas_call(
        functools.partial(_tca_body, blk=blk, d=d),
        grid=grid,
        in_specs=[
            pl.BlockSpec(memory_space=pltpu.SMEM),
            pl.BlockSpec((blk, d1), lambda i: (i, 0)),
            pl.BlockSpec((_NC, blk, d1), lambda i: (0, i, 0)),
            pl.BlockSpec((d, hdim), lambda i: (0, 0)),
            pl.BlockSpec((d, hdim), lambda i: (0, 0)),
            pl.BlockSpec((1, hdim), lambda i: (0, 0)),
            pl.BlockSpec((hdim, cdim), lambda i: (0, 0)),
        ],
        out_specs=[
            pl.BlockSpec((blk, hdim), lambda i: (i, 0)),
            pl.BlockSpec((blk, d2), lambda i: (i, 0)),
        ],
        out_shape=[
            jax.ShapeDtypeStruct((n_pad, hdim), jnp.float32),
            jax.ShapeDtypeStruct((n_pad, d2), jnp.float32),
        ],
    )(res1, xpad, s1, W1_root, W1_neigh, b1.reshape(1, hdim), W2_neigh)

    s2 = _make_sc_segment_sum(n_pad, d2, cpw)(p, edges2, z2)
    s2 = s2.reshape(_NC, n_pad, d2)

    out = pl.pallas_call(
        functools.partial(_tcb_body, blk=blk, c=cdim),
        grid=grid,
        in_specs=[
            pl.BlockSpec(memory_space=pltpu.SMEM),
            pl.BlockSpec((blk, hdim), lambda i: (i, 0)),
            pl.BlockSpec((_NC, blk, d2), lambda i: (0, i, 0)),
            pl.BlockSpec((hdim, cdim), lambda i: (0, 0)),
            pl.BlockSpec((1, cdim), lambda i: (0, 0)),
        ],
        out_specs=pl.BlockSpec((blk, cdim), lambda i: (i, 0)),
        out_shape=jax.ShapeDtypeStruct((n_pad, cdim), jnp.float32),
    )(res2, h, s2, W2_root, b2.reshape(1, cdim))

    return out[:n]


# speedup vs baseline: 1.2953x; 1.2953x over previous
import functools

import jax
import jax.numpy as jnp
from jax import lax
from jax.experimental import pallas as pl
from jax.experimental.pallas import tpu as pltpu
from jax.experimental.pallas import tpu_sc as plsc

_NC = 2
_NS = 16
_CHUNK = 128


def _round_up(v, m):
    return (v + m - 1) // m * m


def _make_sc_segment_sum(n_pad, d_tot, chunks_per_worker):
    nw = _NC * _NS
    rows_per_tile = n_pad // _NS
    epw = chunks_per_worker * _CHUNK
    mesh = plsc.VectorSubcoreMesh(core_axis_name="c", subcore_axis_name="s")

    @functools.partial(
        pl.kernel,
        out_type=jax.ShapeDtypeStruct((_NC * n_pad, d_tot), jnp.float32),
        mesh=mesh,
        compiler_params=pltpu.CompilerParams(use_tc_tiling_on_sc=False),
        scratch_types=[
            pltpu.VMEM((2, _CHUNK), jnp.int32),
            pltpu.VMEM((_CHUNK, d_tot), jnp.float32),
            pltpu.VMEM_SHARED((n_pad, d_tot), jnp.float32),
            pltpu.SemaphoreType.DMA,
        ],
    )
    def sc_kernel(rows_hbm, edges_hbm, zeros_hbm, out_hbm,
                  idx_a, gat_v, acc_sh, sem_g):
        cid = lax.axis_index("c")
        sid = lax.axis_index("s")
        wid = sid * _NC + cid
        r0 = sid * rows_per_tile
        n_sub = rows_per_tile // _CHUNK
        pltpu.sync_copy(zeros_hbm, gat_v)
        for j in range(n_sub):
            pltpu.sync_copy(
                gat_v, acc_sh.at[pl.ds(r0 + j * _CHUNK, _CHUNK)])
        plsc.subcore_barrier()

        def body(i, carry):
            off = wid * epw + i * _CHUNK
            pltpu.sync_copy(edges_hbm.at[:, pl.ds(off, _CHUNK)], idx_a)
            pltpu.async_copy(rows_hbm.at[idx_a.at[0]], gat_v, sem_g).wait()
            pltpu.sync_copy(gat_v, acc_sh.at[idx_a.at[1]], add=True)
            return carry

        lax.fori_loop(0, chunks_per_worker, body, 0)
        plsc.subcore_barrier()
        for j in range(n_sub):
            pltpu.sync_copy(
                acc_sh.at[pl.ds(r0 + j * _CHUNK, _CHUNK)], gat_v)
            pltpu.sync_copy(
                gat_v,
                out_hbm.at[pl.ds(cid * n_pad + r0 + j * _CHUNK, _CHUNK)])

    return sc_kernel


def _tca_body(res1_ref, x_ref, s_ref, w1r_ref, w1n_ref, b1_ref, w2n_ref,
              h_ref, p_ref, *, blk, d):
    i = pl.program_id(0)
    s = s_ref[0] + s_ref[1]
    deg = jnp.maximum(s[:, d:d + 1], 1.0)
    agg = s[:, :d] / deg
    z = (jnp.dot(x_ref[:, :d], w1r_ref[...], preferred_element_type=jnp.float32)
         + jnp.dot(agg, w1n_ref[...], preferred_element_type=jnp.float32)
         + b1_ref[...])
    rows = i * blk + lax.broadcasted_iota(jnp.int32, (blk, 1), 0)
    h = jnp.where(rows < res1_ref[0], jnp.maximum(z, 0.0), 0.0)
    h_ref[...] = h
    p = jnp.dot(h, w2n_ref[...], preferred_element_type=jnp.float32)
    p_ref[...] = jnp.concatenate(
        [p, jnp.ones((blk, 16), jnp.float32)], axis=1)


def _tcb_body(res2_ref, h_ref, s_ref, w2r_ref, b2_ref, o_ref, *, blk, c):
    i = pl.program_id(0)
    s = s_ref[0] + s_ref[1]
    agg = s[:, :c] / jnp.maximum(s[:, c:c + 1], 1.0)
    z = (jnp.dot(h_ref[...], w2r_ref[...], preferred_element_type=jnp.float32)
         + agg + b2_ref[...])
    rows = i * blk + lax.broadcasted_iota(jnp.int32, (blk, 1), 0)
    z = jnp.where(rows < res2_ref[0], z, 0.0)
    m = jnp.max(z, axis=1, keepdims=True)
    e = jnp.exp(z - m)
    o_ref[...] = z - m - jnp.log(jnp.sum(e, axis=1, keepdims=True))


def kernel(x, edge_index1, res_size1, edge_index2, res_size2,
           W1_root, W1_neigh, b1, W2_root, W2_neigh, b2):
    n, d = x.shape
    hdim = W1_root.shape[1]
    cdim = W2_root.shape[1]
    e = edge_index1.shape[1]
    blk = 512
    n_pad = _round_up(n + 1, blk)
    nw = _NC * _NS
    cpw = -(-e // (nw * _CHUNK))
    e_pad = cpw * nw * _CHUNK
    d1 = d + 16
    d2 = cdim + 16

    xpad = jnp.pad(
        jnp.concatenate([x, jnp.ones((n, 16), jnp.float32)], axis=1),
        ((0, n_pad - n), (0, 0)))
    ei1 = edge_index1.astype(jnp.int32)
    ei2 = edge_index2.astype(jnp.int32)
    trash = n + jnp.arange(e_pad - e, dtype=jnp.int32) % (n_pad - n)
    edges1 = jnp.stack([jnp.pad(ei1[0], (0, e_pad - e)),
                        jnp.concatenate([ei1[1], trash])])
    edges2 = jnp.stack([jnp.pad(ei2[0], (0, e_pad - e)),
                        jnp.concatenate([ei2[1], trash])])
    z1 = jnp.zeros((_CHUNK, d1), jnp.float32)
    z2 = jnp.zeros((_CHUNK, d2), jnp.float32)
    res1 = jnp.asarray(res_size1, jnp.int32).reshape(1)
    res2 = jnp.asarray(res_size2, jnp.int32).reshape(1)

    s1 = _make_sc_segment_sum(n_pad, d1, cpw)(xpad, edges1, z1)
    s1 = s1.reshape(_NC, n_pad, d1)

    grid = (n_pad // blk,)
    h, p = pl.pallas_call(
        functools.partial(_tca_body, blk=blk, d=d),
        grid=grid,
        in_specs=[
            pl.BlockSpec(memory_space=pltpu.SMEM),
            pl.BlockSpec((blk, d1), lambda i: (i, 0)),
            pl.BlockSpec((_NC, blk, d1), lambda i: (0, i, 0)),
            pl.BlockSpec((d, hdim), lambda i: (0, 0)),
            pl.BlockSpec((d, hdim), lambda i: (0, 0)),
            pl.BlockSpec((1, hdim), lambda i: (0, 0)),
            pl.BlockSpec((hdim, cdim), lambda i: (0, 0)),
        ],
        out_specs=[
            pl.BlockSpec((blk, hdim), lambda i: (i, 0)),
            pl.BlockSpec((blk, d2), lambda i: (i, 0)),
        ],
        out_shape=[
            jax.ShapeDtypeStruct((n_pad, hdim), jnp.float32),
            jax.ShapeDtypeStruct((n_pad, d2), jnp.float32),
        ],
    )(res1, xpad, s1, W1_root, W1_neigh, b1.reshape(1, hdim), W2_neigh)

    s2 = _make_sc_segment_sum(n_pad, d2, cpw)(p, edges2, z2)
    s2 = s2.reshape(_NC, n_pad, d2)

    out = pl.pallas_call(
        functools.partial(_tcb_body, blk=blk, c=cdim),
        grid=grid,
        in_specs=[
            pl.BlockSpec(memory_space=pltpu.SMEM),
            pl.BlockSpec((blk, hdim), lambda i: (i, 0)),
            pl.BlockSpec((_NC, blk, d2), lambda i: (0, i, 0)),
            pl.BlockSpec((hdim, cdim), lambda i: (0, 0)),
            pl.BlockSpec((1, cdim), lambda i: (0, 0)),
        ],
        out_specs=pl.BlockSpec((blk, cdim), lambda i: (i, 0)),
        out_shape=jax.ShapeDtypeStruct((n_pad, cdim), jnp.float32),
    )(res2, h, s2, W2_root, b2.reshape(1, cdim))

    return out[:n]


# speedup vs baseline: 1.3815x; 1.0665x over previous
import functools

import jax
import jax.numpy as jnp
from jax import lax
from jax.experimental import pallas as pl
from jax.experimental.pallas import tpu as pltpu
from jax.experimental.pallas import tpu_sc as plsc

_NC = 2
_NS = 16
_CHUNK = 128


def _round_up(v, m):
    return (v + m - 1) // m * m


def _make_sc_segment_sum(n_pad, d_tot, chunks_per_worker):
    nw = _NC * _NS
    rows_per_tile = n_pad // _NS
    epw = chunks_per_worker * _CHUNK
    mesh = plsc.VectorSubcoreMesh(core_axis_name="c", subcore_axis_name="s")

    @functools.partial(
        pl.kernel,
        out_type=jax.ShapeDtypeStruct((_NC * n_pad, d_tot), jnp.float32),
        mesh=mesh,
        compiler_params=pltpu.CompilerParams(use_tc_tiling_on_sc=False),
        scratch_types=[
            pltpu.VMEM((2, _CHUNK), jnp.int32),
            pltpu.VMEM((_CHUNK, d_tot), jnp.float32),
            pltpu.VMEM_SHARED((n_pad, d_tot), jnp.float32),
            pltpu.SemaphoreType.DMA,
        ],
    )
    def sc_kernel(rows_hbm, edges_hbm, zeros_hbm, out_hbm,
                  idx_a, gat_v, acc_sh, sem_g):
        cid = lax.axis_index("c")
        sid = lax.axis_index("s")
        wid = sid * _NC + cid
        r0 = sid * rows_per_tile
        n_sub = rows_per_tile // _CHUNK
        pltpu.sync_copy(zeros_hbm, gat_v)
        for j in range(n_sub):
            pltpu.sync_copy(
                gat_v, acc_sh.at[pl.ds(r0 + j * _CHUNK, _CHUNK)])
        plsc.subcore_barrier()

        def body(i, carry):
            off = wid * epw + i * _CHUNK
            pltpu.sync_copy(edges_hbm.at[:, pl.ds(off, _CHUNK)], idx_a)
            pltpu.async_copy(rows_hbm.at[idx_a.at[0]], gat_v, sem_g).wait()
            pltpu.sync_copy(gat_v, acc_sh.at[idx_a.at[1]], add=True)
            return carry

        lax.fori_loop(0, chunks_per_worker, body, 0)
        plsc.subcore_barrier()
        for j in range(n_sub):
            pltpu.sync_copy(
                acc_sh.at[pl.ds(r0 + j * _CHUNK, _CHUNK)], gat_v)
            pltpu.sync_copy(
                gat_v,
                out_hbm.at[pl.ds(cid * n_pad + r0 + j * _CHUNK, _CHUNK)])

    return sc_kernel


def _tca_body(res1_ref, x_ref, s_ref, w1r_ref, w1n_ref, b1_ref, w2n_ref,
              h_ref, p_ref, *, blk, d):
    i = pl.program_id(0)
    s = s_ref[0] + s_ref[1]
    deg = jnp.maximum(s[:, d:d + 1], 1.0)
    agg = s[:, :d] / deg
    z = (jnp.dot(x_ref[:, :d], w1r_ref[...], preferred_element_type=jnp.float32)
         + jnp.dot(agg, w1n_ref[...], preferred_element_type=jnp.float32)
         + b1_ref[...])
    rows = i * blk + lax.broadcasted_iota(jnp.int32, (blk, 1), 0)
    h = jnp.where(rows < res1_ref[0], jnp.maximum(z, 0.0), 0.0)
    h_ref[...] = h
    p = jnp.dot(h, w2n_ref[...], preferred_element_type=jnp.float32)
    p_ref[...] = jnp.concatenate(
        [p, jnp.ones((blk, 16), jnp.float32)], axis=1)


def _tcb_body(res2_ref, h_ref, s_ref, w2r_ref, b2_ref, o_ref, *, blk, c):
    i = pl.program_id(0)
    s = s_ref[0] + s_ref[1]
    agg = s[:, :c] / jnp.maximum(s[:, c:c + 1], 1.0)
    z = (jnp.dot(h_ref[...], w2r_ref[...], preferred_element_type=jnp.float32)
         + agg + b2_ref[...])
    rows = i * blk + lax.broadcasted_iota(jnp.int32, (blk, 1), 0)
    z = jnp.where(rows < res2_ref[0], z, 0.0)
    m = jnp.max(z, axis=1, keepdims=True)
    e = jnp.exp(z - m)
    o_ref[...] = z - m - jnp.log(jnp.sum(e, axis=1, keepdims=True))


def kernel(x, edge_index1, res_size1, edge_index2, res_size2,
           W1_root, W1_neigh, b1, W2_root, W2_neigh, b2):
    n, d = x.shape
    hdim = W1_root.shape[1]
    cdim = W2_root.shape[1]
    e = edge_index1.shape[1]
    blk = 512
    n_pad = _round_up(n + 1, blk)
    nw = _NC * _NS
    cpw = -(-e // (nw * _CHUNK))
    e_pad = cpw * nw * _CHUNK
    d1 = d + 16
    d2 = cdim + 16

    xpad = jnp.pad(
        jnp.concatenate([x, jnp.ones((n, 16), jnp.float32)], axis=1),
        ((0, n_pad - n), (0, 0)))
    ei1 = edge_index1.astype(jnp.int32)
    ei2 = edge_index2.astype(jnp.int32)
    edges1 = jnp.stack([jnp.pad(ei1[0], (0, e_pad - e)),
                        jnp.pad(ei1[1], (0, e_pad - e), constant_values=n)])
    edges2 = jnp.stack([jnp.pad(ei2[0], (0, e_pad - e)),
                        jnp.pad(ei2[1], (0, e_pad - e), constant_values=n)])
    z1 = jnp.zeros((_CHUNK, d1), jnp.float32)
    z2 = jnp.zeros((_CHUNK, d2), jnp.float32)
    res1 = jnp.asarray(res_size1, jnp.int32).reshape(1)
    res2 = jnp.asarray(res_size2, jnp.int32).reshape(1)

    s1 = _make_sc_segment_sum(n_pad, d1, cpw)(xpad, edges1, z1)
    s1 = s1.reshape(_NC, n_pad, d1)

    grid = (n_pad // blk,)
    h, p = pl.pallas_call(
        functools.partial(_tca_body, blk=blk, d=d),
        grid=grid,
        in_specs=[
            pl.BlockSpec(memory_space=pltpu.SMEM),
            pl.BlockSpec((blk, d1), lambda i: (i, 0)),
            pl.BlockSpec((_NC, blk, d1), lambda i: (0, i, 0)),
            pl.BlockSpec((d, hdim), lambda i: (0, 0)),
            pl.BlockSpec((d, hdim), lambda i: (0, 0)),
            pl.BlockSpec((1, hdim), lambda i: (0, 0)),
            pl.BlockSpec((hdim, cdim), lambda i: (0, 0)),
        ],
        out_specs=[
            pl.BlockSpec((blk, hdim), lambda i: (i, 0)),
            pl.BlockSpec((blk, d2), lambda i: (i, 0)),
        ],
        out_shape=[
            jax.ShapeDtypeStruct((n_pad, hdim), jnp.float32),
            jax.ShapeDtypeStruct((n_pad, d2), jnp.float32),
        ],
    )(res1, xpad, s1, W1_root, W1_neigh, b1.reshape(1, hdim), W2_neigh)

    s2 = _make_sc_segment_sum(n_pad, d2, cpw)(p, edges2, z2)
    s2 = s2.reshape(_NC, n_pad, d2)

    out = pl.pallas_call(
        functools.partial(_tcb_body, blk=blk, c=cdim),
        grid=grid,
        in_specs=[
            pl.BlockSpec(memory_space=pltpu.SMEM),
            pl.BlockSpec((blk, hdim), lambda i: (i, 0)),
            pl.BlockSpec((_NC, blk, d2), lambda i: (0, i, 0)),
            pl.BlockSpec((hdim, cdim), lambda i: (0, 0)),
            pl.BlockSpec((1, cdim), lambda i: (0, 0)),
        ],
        out_specs=pl.BlockSpec((blk, cdim), lambda i: (i, 0)),
        out_shape=jax.ShapeDtypeStruct((n_pad, cdim), jnp.float32),
    )(res2, h, s2, W2_root, b2.reshape(1, cdim))

    return out[:n]


# speedup vs baseline: 1.6969x; 1.2283x over previous
import functools

import jax
import jax.numpy as jnp
from jax import lax
from jax.experimental import pallas as pl
from jax.experimental.pallas import tpu as pltpu
from jax.experimental.pallas import tpu_sc as plsc

_NC = 2
_NS = 16
_CHUNK = 128
_F0_L1 = 0.61
_F0_L2 = 0.61


def _round_up(v, m):
    return (v + m - 1) // m * m


def _make_sc_segment_sum(n_pad, d_tot, cpw0, cpw1):
    rows_per_tile = n_pad // _NS
    n0 = _NS * cpw0
    mesh = plsc.VectorSubcoreMesh(core_axis_name="c", subcore_axis_name="s")

    @functools.partial(
        pl.kernel,
        out_type=jax.ShapeDtypeStruct((_NC * n_pad, d_tot), jnp.float32),
        mesh=mesh,
        compiler_params=pltpu.CompilerParams(use_tc_tiling_on_sc=False),
        scratch_types=[
            pltpu.VMEM((2, _CHUNK), jnp.int32),
            pltpu.VMEM((_CHUNK, d_tot), jnp.float32),
            pltpu.VMEM_SHARED((n_pad, d_tot), jnp.float32),
            pltpu.SemaphoreType.DMA,
        ],
    )
    def sc_kernel(rows_hbm, edges_hbm, zeros_hbm, out_hbm,
                  idx_a, gat_v, acc_sh, sem_g):
        cid = lax.axis_index("c")
        sid = lax.axis_index("s")
        r0 = sid * rows_per_tile
        n_sub = rows_per_tile // _CHUNK
        pltpu.sync_copy(zeros_hbm, gat_v)
        for j in range(n_sub):
            pltpu.sync_copy(
                gat_v, acc_sh.at[pl.ds(r0 + j * _CHUNK, _CHUNK)])
        plsc.subcore_barrier()

        def step(chunk_idx):
            off = chunk_idx * _CHUNK
            pltpu.sync_copy(edges_hbm.at[:, pl.ds(off, _CHUNK)], idx_a)
            pltpu.async_copy(rows_hbm.at[idx_a.at[0]], gat_v, sem_g).wait()
            pltpu.sync_copy(gat_v, acc_sh.at[idx_a.at[1]], add=True)

        @pl.when(cid == 0)
        def _():
            lax.fori_loop(
                0, cpw0, lambda i, c: (step(sid * cpw0 + i), c)[1], 0)

        @pl.when(cid == 1)
        def _():
            lax.fori_loop(
                0, cpw1, lambda i, c: (step(n0 + sid * cpw1 + i), c)[1], 0)
        plsc.subcore_barrier()
        for j in range(n_sub):
            pltpu.sync_copy(
                acc_sh.at[pl.ds(r0 + j * _CHUNK, _CHUNK)], gat_v)
            pltpu.sync_copy(
                gat_v,
                out_hbm.at[pl.ds(cid * n_pad + r0 + j * _CHUNK, _CHUNK)])

    return sc_kernel


def _tca_body(res1_ref, x_ref, s_ref, w1r_ref, w1n_ref, b1_ref, w2n_ref,
              h_ref, p_ref, *, blk, d):
    i = pl.program_id(0)
    s = s_ref[0] + s_ref[1]
    deg = jnp.maximum(s[:, d:d + 1], 1.0)
    agg = s[:, :d] / deg
    z = (jnp.dot(x_ref[:, :d], w1r_ref[...], preferred_element_type=jnp.float32)
         + jnp.dot(agg, w1n_ref[...], preferred_element_type=jnp.float32)
         + b1_ref[...])
    rows = i * blk + lax.broadcasted_iota(jnp.int32, (blk, 1), 0)
    h = jnp.where(rows < res1_ref[0], jnp.maximum(z, 0.0), 0.0)
    h_ref[...] = h
    p = jnp.dot(h, w2n_ref[...], preferred_element_type=jnp.float32)
    p_ref[...] = jnp.concatenate(
        [p, jnp.ones((blk, 16), jnp.float32)], axis=1)


def _tcb_body(res2_ref, h_ref, s_ref, w2r_ref, b2_ref, o_ref, *, blk, c):
    i = pl.program_id(0)
    s = s_ref[0] + s_ref[1]
    agg = s[:, :c] / jnp.maximum(s[:, c:c + 1], 1.0)
    z = (jnp.dot(h_ref[...], w2r_ref[...], preferred_element_type=jnp.float32)
         + agg + b2_ref[...])
    rows = i * blk + lax.broadcasted_iota(jnp.int32, (blk, 1), 0)
    z = jnp.where(rows < res2_ref[0], z, 0.0)
    m = jnp.max(z, axis=1, keepdims=True)
    e = jnp.exp(z - m)
    o_ref[...] = z - m - jnp.log(jnp.sum(e, axis=1, keepdims=True))


def kernel(x, edge_index1, res_size1, edge_index2, res_size2,
           W1_root, W1_neigh, b1, W2_root, W2_neigh, b2):
    n, d = x.shape
    hdim = W1_root.shape[1]
    cdim = W2_root.shape[1]
    e = edge_index1.shape[1]
    blk = 512
    n_pad = _round_up(n + 1, blk)
    tot = -(-e // (_NS * _CHUNK))
    cpw0a = max(1, min(tot - 1, round(tot * _F0_L1)))
    cpw0b = max(1, min(tot - 1, round(tot * _F0_L2)))
    e_pad = tot * _NS * _CHUNK
    d1 = d + 16
    d2 = cdim + 16

    xpad = jnp.pad(
        jnp.concatenate([x, jnp.ones((n, 16), jnp.float32)], axis=1),
        ((0, n_pad - n), (0, 0)))
    ei1 = edge_index1.astype(jnp.int32)
    ei2 = edge_index2.astype(jnp.int32)
    edges1 = jnp.stack([jnp.pad(ei1[0], (0, e_pad - e)),
                        jnp.pad(ei1[1], (0, e_pad - e), constant_values=n)])
    edges2 = jnp.stack([jnp.pad(ei2[0], (0, e_pad - e)),
                        jnp.pad(ei2[1], (0, e_pad - e), constant_values=n)])
    z1 = jnp.zeros((_CHUNK, d1), jnp.float32)
    z2 = jnp.zeros((_CHUNK, d2), jnp.float32)
    res1 = jnp.asarray(res_size1, jnp.int32).reshape(1)
    res2 = jnp.asarray(res_size2, jnp.int32).reshape(1)

    s1 = _make_sc_segment_sum(n_pad, d1, cpw0a, tot - cpw0a)(xpad, edges1, z1)
    s1 = s1.reshape(_NC, n_pad, d1)

    grid = (n_pad // blk,)
    h, p = pl.pallas_call(
        functools.partial(_tca_body, blk=blk, d=d),
        grid=grid,
        in_specs=[
            pl.BlockSpec(memory_space=pltpu.SMEM),
            pl.BlockSpec((blk, d1), lambda i: (i, 0)),
            pl.BlockSpec((_NC, blk, d1), lambda i: (0, i, 0)),
            pl.BlockSpec((d, hdim), lambda i: (0, 0)),
            pl.BlockSpec((d, hdim), lambda i: (0, 0)),
            pl.BlockSpec((1, hdim), lambda i: (0, 0)),
            pl.BlockSpec((hdim, cdim), lambda i: (0, 0)),
        ],
        out_specs=[
            pl.BlockSpec((blk, hdim), lambda i: (i, 0)),
            pl.BlockSpec((blk, d2), lambda i: (i, 0)),
        ],
        out_shape=[
            jax.ShapeDtypeStruct((n_pad, hdim), jnp.float32),
            jax.ShapeDtypeStruct((n_pad, d2), jnp.float32),
        ],
    )(res1, xpad, s1, W1_root, W1_neigh, b1.reshape(1, hdim), W2_neigh)

    s2 = _make_sc_segment_sum(n_pad, d2, cpw0b, tot - cpw0b)(p, edges2, z2)
    s2 = s2.reshape(_NC, n_pad, d2)

    out = pl.pallas_call(
        functools.partial(_tcb_body, blk=blk, c=cdim),
        grid=grid,
        in_specs=[
            pl.BlockSpec(memory_space=pltpu.SMEM),
            pl.BlockSpec((blk, hdim), lambda i: (i, 0)),
            pl.BlockSpec((_NC, blk, d2), lambda i: (0, i, 0)),
            pl.BlockSpec((hdim, cdim), lambda i: (0, 0)),
            pl.BlockSpec((1, cdim), lambda i: (0, 0)),
        ],
        out_specs=pl.BlockSpec((blk, cdim), lambda i: (i, 0)),
        out_shape=jax.ShapeDtypeStruct((n_pad, cdim), jnp.float32),
    )(res2, h, s2, W2_root, b2.reshape(1, cdim))

    return out[:n]
